# baseline (speedup 1.0000x reference)
import functools

import jax
import jax.numpy as jnp
from jax import lax
from jax.experimental import pallas as pl
from jax.experimental.pallas import tpu as pltpu
from jax.experimental.pallas import tpu_sc as plsc

_B, _C, _H, _W = 8, 19, 512, 512
_NPIX = _B * _H * _W
_NW = 32
_P = _NPIX // _NW
_K = 2048
_NCH = _P // _K
_HW = _H * _W


def _neg_log(x):
    bits = lax.bitcast_convert_type(x, jnp.int32)
    e = lax.shift_right_arithmetic(bits, 23) - 126
    m = lax.bitcast_convert_type(
        jnp.bitwise_or(jnp.bitwise_and(bits, 0x007FFFFF), 0x3F000000),
        jnp.float32)
    small = m < jnp.float32(0.70710678)
    e = e - jnp.where(small, 1, 0)
    f = jnp.where(small, m + m, m) - jnp.float32(1.0)
    z = f * f
    y = jnp.full((16,), 7.0376836292e-2, jnp.float32)
    for c in (-1.1514610310e-1, 1.1676998740e-1, -1.2420140846e-1,
              1.4249322787e-1, -1.6668057665e-1, 2.0000714765e-1,
              -2.4999993993e-1, 3.3333331174e-1):
        y = y * f + jnp.float32(c)
    y = y * f * z
    ef = e.astype(jnp.float32)
    y = y + ef * jnp.float32(-2.12194440e-4)
    y = y - jnp.float32(0.5) * z
    return -(f + y + ef * jnp.float32(0.693359375))


_mesh = plsc.VectorSubcoreMesh(core_axis_name="c", subcore_axis_name="s")


@functools.partial(
    pl.kernel,
    out_type=(jax.ShapeDtypeStruct((_NW, 16), jnp.float32),
              jax.ShapeDtypeStruct((_NW, 16), jnp.float32)),
    mesh=_mesh,
    scratch_types=[
        pltpu.VMEM((_K,), jnp.int32),
        pltpu.VMEM((_K,), jnp.int32),
        pltpu.VMEM((_K,), jnp.float32),
        pltpu.VMEM((16,), jnp.float32),
        pltpu.VMEM((16,), jnp.float32),
        pltpu.SemaphoreType.DMA,
    ],
)
def _nll_sc(score_hbm, tgt_hbm, sum_out, cnt_out,
            tgt_v, idx_v, vals_v, acc_v, cnt_v, sem):
    wid = lax.axis_index("s") * 2 + lax.axis_index("c")
    base = wid * _P
    lanes = lax.broadcasted_iota(jnp.int32, (16,), 0)

    def chunk(ci, carry):
        acc, cnt = carry
        cbase = base + ci * _K
        pltpu.sync_copy(tgt_hbm.at[pl.ds(cbase, _K)], tgt_v)

        def idx_body(i, _):
            t = tgt_v[pl.ds(i * 16, 16)]
            p = cbase + i * 16 + lanes
            b = lax.shift_right_logical(p, 18)
            idx_v[pl.ds(i * 16, 16)] = p + lax.shift_left(b * 18 + t, 18)
            return 0

        lax.fori_loop(0, _K // 16, idx_body, 0)
        pltpu.async_copy(score_hbm.at[idx_v], vals_v, sem).wait()

        def red_body(i, c):
            a, n = c
            nl = _neg_log(vals_v[pl.ds(i * 16, 16)])
            return (a + nl,
                    n + jnp.where(nl != 0.0, jnp.float32(1.0), jnp.float32(0.0)))

        return lax.fori_loop(0, _K // 16, red_body, (acc, cnt))

    zero = jnp.zeros((16,), jnp.float32)
    acc, cnt = lax.fori_loop(0, _NCH, chunk, (zero, zero))
    acc_v[...] = acc
    cnt_v[...] = cnt
    pltpu.sync_copy(acc_v, sum_out.at[wid])
    pltpu.sync_copy(cnt_v, cnt_out.at[wid])


def kernel(score, target):
    sums, cnts = _nll_sc(score.reshape(-1), target.reshape(-1))
    return jnp.sum(sums) / jnp.sum(cnts)


# speedup vs baseline: 3.1554x; 3.1554x over previous
import functools

import jax
import jax.numpy as jnp
from jax import lax
from jax.experimental import pallas as pl
from jax.experimental.pallas import tpu as pltpu
from jax.experimental.pallas import tpu_sc as plsc

_B, _C, _H, _W = 8, 19, 512, 512
_NW = 32
_NBLK = _B * (_H // 8) * (_W // 128)
_BPW = _NBLK // _NW


def _neg_log(x):
    bits = lax.bitcast_convert_type(x, jnp.int32)
    e = lax.shift_right_arithmetic(bits, 23) - 126
    m = lax.bitcast_convert_type(
        jnp.bitwise_or(jnp.bitwise_and(bits, 0x007FFFFF), 0x3F000000),
        jnp.float32)
    small = m < jnp.float32(0.70710678)
    e = e - jnp.where(small, 1, 0)
    f = jnp.where(small, m + m, m) - jnp.float32(1.0)
    z = f * f
    y = jnp.full((16,), 7.0376836292e-2, jnp.float32)
    for c in (-1.1514610310e-1, 1.1676998740e-1, -1.2420140846e-1,
              1.4249322787e-1, -1.6668057665e-1, 2.0000714765e-1,
              -2.4999993993e-1, 3.3333331174e-1):
        y = y * f + jnp.float32(c)
    y = y * f * z
    ef = e.astype(jnp.float32)
    y = y + ef * jnp.float32(-2.12194440e-4)
    y = y - jnp.float32(0.5) * z
    return -(f + y + ef * jnp.float32(0.693359375))


_mesh = plsc.VectorSubcoreMesh(core_axis_name="c", subcore_axis_name="s")


@functools.partial(
    pl.kernel,
    out_type=(jax.ShapeDtypeStruct((_NW, 16), jnp.float32),
              jax.ShapeDtypeStruct((_NW, 16), jnp.float32)),
    mesh=_mesh,
    scratch_types=[
        pltpu.VMEM((2 * _C * 8, 128), jnp.float32),
        pltpu.VMEM((2, 8, 128), jnp.int32),
        pltpu.VMEM((16,), jnp.float32),
        pltpu.VMEM((16,), jnp.float32),
        pltpu.SemaphoreType.DMA,
    ],
    compiler_params=pltpu.CompilerParams(needs_layout_passes=False),
)
def _nll_sc(score_4d, tgt_3d, sum_out, cnt_out,
            cls_v, tgt_v, acc_v, cnt_v, sem):
    wid = lax.axis_index("s") * 2 + lax.axis_index("c")
    lanes = lax.broadcasted_iota(jnp.int32, (16,), 0)

    def _descs(bi):
        g = wid * _BPW + bi
        b = lax.shift_right_logical(g, 8)
        h0 = lax.bitwise_and(lax.shift_right_logical(g, 2), 63) * 8
        w0 = lax.bitwise_and(g, 3) * 128
        par = lax.bitwise_and(bi, 1)
        ds = [pltpu.make_async_copy(
                  score_4d.at[b, c, pl.ds(h0, 8), pl.ds(w0, 128)],
                  cls_v.at[pl.ds((par * _C + c) * 8, 8)], sem)
              for c in range(_C)]
        ds.append(pltpu.make_async_copy(
            tgt_3d.at[b, pl.ds(h0, 8), pl.ds(w0, 128)], tgt_v.at[par], sem))
        return ds

    def block(bi, carry):
        acc, cnt = carry

        @pl.when(bi + 1 < _BPW)
        def _():
            for d in _descs(bi + 1):
                d.start()

        for d in _descs(bi):
            d.wait()

        par = lax.bitwise_and(bi, 1)
        row_base = par * (_C * 8)

        def red_body(g, c):
            a, n = c
            hl = lax.shift_right_logical(g, 3)
            wj = lax.bitwise_and(g, 7)
            t = tgt_v[par, hl, pl.ds(wj * 16, 16)]
            v = plsc.load_gather(
                cls_v, [row_base + t * 8 + hl, wj * 16 + lanes])
            nl = _neg_log(v)
            return (a + nl,
                    n + jnp.where(nl != 0.0, jnp.float32(1.0), jnp.float32(0.0)))

        return lax.fori_loop(0, 64, red_body, (acc, cnt))

    for d in _descs(0):
        d.start()
    zero = jnp.zeros((16,), jnp.float32)
    acc, cnt = lax.fori_loop(0, _BPW, block, (zero, zero))
    acc_v[...] = acc
    cnt_v[...] = cnt
    pltpu.sync_copy(acc_v, sum_out.at[wid])
    pltpu.sync_copy(cnt_v, cnt_out.at[wid])


def kernel(score, target):
    sums, cnts = _nll_sc(score, target)
    return jnp.sum(sums) / jnp.sum(cnts)


# speedup vs baseline: 3.2993x; 1.0456x over previous
import functools

import jax
import jax.numpy as jnp
from jax import lax
from jax.experimental import pallas as pl
from jax.experimental.pallas import tpu as pltpu
from jax.experimental.pallas import tpu_sc as plsc

_B, _C, _H, _W = 8, 19, 512, 512
_NW = 32
_NBLK = _B * (_H // 8) * (_W // 128)
_BPW = _NBLK // _NW


def _log(x):
    bits = lax.bitcast_convert_type(x, jnp.int32)
    e = lax.shift_right_arithmetic(bits, 23) - 126
    m = lax.bitcast_convert_type(
        jnp.bitwise_or(jnp.bitwise_and(bits, 0x007FFFFF), 0x3F000000),
        jnp.float32)
    small = m < jnp.float32(0.70710678)
    e = e - jnp.where(small, 1, 0)
    f = jnp.where(small, m + m, m) - jnp.float32(1.0)
    z = f * f
    y = jnp.full((16,), 1.4249322787e-1, jnp.float32)
    for c in (-1.6668057665e-1, 2.0000714765e-1,
              -2.4999993993e-1, 3.3333331174e-1):
        y = y * f + jnp.float32(c)
    y = y * f * z
    y = y - jnp.float32(0.5) * z
    return f + y + e.astype(jnp.float32) * jnp.float32(0.6931471805599453)


_mesh = plsc.VectorSubcoreMesh(core_axis_name="c", subcore_axis_name="s")


@functools.partial(
    pl.kernel,
    out_type=(jax.ShapeDtypeStruct((_NW, 16), jnp.float32),
              jax.ShapeDtypeStruct((_NW, 16), jnp.float32)),
    mesh=_mesh,
    scratch_types=[
        pltpu.VMEM((2 * _C * 8, 128), jnp.float32),
        pltpu.VMEM((2, 8, 128), jnp.int32),
        pltpu.VMEM((16,), jnp.float32),
        pltpu.VMEM((16,), jnp.float32),
        pltpu.SemaphoreType.DMA,
    ],
    compiler_params=pltpu.CompilerParams(needs_layout_passes=False),
)
def _nll_sc(score_4d, tgt_3d, sum_out, cnt_out,
            cls_v, tgt_v, acc_v, cnt_v, sem):
    wid = lax.axis_index("s") * 2 + lax.axis_index("c")
    lanes = lax.broadcasted_iota(jnp.int32, (16,), 0)

    def _descs(bi):
        g = wid * _BPW + bi
        b = lax.shift_right_logical(g, 8)
        h0 = lax.bitwise_and(lax.shift_right_logical(g, 2), 63) * 8
        w0 = lax.bitwise_and(g, 3) * 128
        par = lax.bitwise_and(bi, 1)
        ds = [pltpu.make_async_copy(
                  score_4d.at[b, c, pl.ds(h0, 8), pl.ds(w0, 128)],
                  cls_v.at[pl.ds((par * _C + c) * 8, 8)], sem)
              for c in range(_C)]
        ds.append(pltpu.make_async_copy(
            tgt_3d.at[b, pl.ds(h0, 8), pl.ds(w0, 128)], tgt_v.at[par], sem))
        return ds

    def block(bi, carry):
        acc, cnt = carry

        @pl.when(bi + 1 < _BPW)
        def _():
            for d in _descs(bi + 1):
                d.start()

        for d in _descs(bi):
            d.wait()

        par = lax.bitwise_and(bi, 1)
        row_base = par * (_C * 8)

        @plsc.parallel_loop(0, 64, carry=(acc, cnt), unroll=8)
        def red_body(g, c):
            a, n = c
            hl = lax.shift_right_logical(g, 3)
            wj = lax.bitwise_and(g, 7)
            t = tgt_v[par, hl, pl.ds(wj * 16, 16)]
            v = plsc.load_gather(
                cls_v, [(row_base + hl) + t * 8, wj * 16 + lanes])
            lg = _log(v)
            return (a + lg,
                    n + jnp.where(lg != 0.0, jnp.float32(1.0), jnp.float32(0.0)))

        return red_body

    for d in _descs(0):
        d.start()
    zero = jnp.zeros((16,), jnp.float32)
    acc, cnt = lax.fori_loop(0, _BPW, block, (zero, zero))
    acc_v[...] = acc
    cnt_v[...] = cnt
    pltpu.sync_copy(acc_v, sum_out.at[wid])
    pltpu.sync_copy(cnt_v, cnt_out.at[wid])


def kernel(score, target):
    sums, cnts = _nll_sc(score, target)
    return -jnp.sum(sums) / jnp.sum(cnts)


# speedup vs baseline: 3.3759x; 1.0232x over previous
import functools

import jax
import jax.numpy as jnp
from jax import lax
from jax.experimental import pallas as pl
from jax.experimental.pallas import tpu as pltpu
from jax.experimental.pallas import tpu_sc as plsc

_B, _C, _H, _W = 8, 19, 512, 512
_NW = 32
_NBLK = _B * (_H // 8) * (_W // 128)
_BPW = _NBLK // _NW
_SQRTHF_BITS = 0x3F3504F3


def _log_parts(v):
    bits = lax.bitcast_convert_type(v, jnp.int32)
    bias = bits - _SQRTHF_BITS
    e = lax.shift_right_arithmetic(bias, 23)
    m = lax.bitcast_convert_type(
        jnp.bitwise_and(bias, 0x007FFFFF) + _SQRTHF_BITS, jnp.float32)
    f = m - jnp.float32(1.0)
    z = f * f
    y = f * jnp.float32(-1.6668057665e-1) + jnp.float32(2.0000714765e-1)
    y = y * f + jnp.float32(-2.4999993993e-1)
    y = y * f + jnp.float32(3.3333331174e-1)
    y = y * f * z - jnp.float32(0.5) * z
    return f + y, e


_mesh = plsc.VectorSubcoreMesh(core_axis_name="c", subcore_axis_name="s")


@functools.partial(
    pl.kernel,
    out_type=(jax.ShapeDtypeStruct((_NW, 16), jnp.float32),
              jax.ShapeDtypeStruct((_NW, 16), jnp.int32),
              jax.ShapeDtypeStruct((_NW, 16), jnp.float32)),
    mesh=_mesh,
    scratch_types=[
        pltpu.VMEM((2 * _C * 8, 128), jnp.float32),
        pltpu.VMEM((2, 8, 128), jnp.int32),
        pltpu.VMEM((16,), jnp.float32),
        pltpu.VMEM((16,), jnp.int32),
        pltpu.VMEM((16,), jnp.float32),
        pltpu.SemaphoreType.DMA,
    ],
    compiler_params=pltpu.CompilerParams(needs_layout_passes=False),
)
def _nll_sc(score_4d, tgt_3d, fsum_out, esum_out, cnt_out,
            cls_v, tgt_v, fs_v, es_v, cn_v, sem):
    wid = lax.axis_index("s") * 2 + lax.axis_index("c")
    lanes = lax.broadcasted_iota(jnp.int32, (16,), 0)

    def _descs(bi):
        g = wid * _BPW + bi
        b = lax.shift_right_logical(g, 8)
        h0 = lax.bitwise_and(lax.shift_right_logical(g, 2), 63) * 8
        w0 = lax.bitwise_and(g, 3) * 128
        par = lax.bitwise_and(bi, 1)
        ds = [pltpu.make_async_copy(
                  score_4d.at[b, c, pl.ds(h0, 8), pl.ds(w0, 128)],
                  cls_v.at[pl.ds((par * _C + c) * 8, 8)], sem)
              for c in range(_C)]
        ds.append(pltpu.make_async_copy(
            tgt_3d.at[b, pl.ds(h0, 8), pl.ds(w0, 128)], tgt_v.at[par], sem))
        return ds

    def block(bi, carry):
        @pl.when(bi + 1 < _BPW)
        def _():
            for d in _descs(bi + 1):
                d.start()

        for d in _descs(bi):
            d.wait()

        par = lax.bitwise_and(bi, 1)
        row_base = par * (_C * 8)

        @plsc.parallel_loop(0, 64, carry=carry, unroll=8)
        def red_body(g, c):
            afy, ae, n = c
            hl = lax.shift_right_logical(g, 3)
            wj = lax.bitwise_and(g, 7)
            t = tgt_v[par, hl, pl.ds(wj * 16, 16)]
            v = plsc.load_gather(
                cls_v, [(row_base + hl) + t * 8, wj * 16 + lanes])
            fy, e = _log_parts(v)
            return (afy + fy, ae + e,
                    n + jnp.where(v != jnp.float32(1.0),
                                  jnp.float32(1.0), jnp.float32(0.0)))

        return red_body

    for d in _descs(0):
        d.start()
    zf = jnp.zeros((16,), jnp.float32)
    zi = jnp.zeros((16,), jnp.int32)
    afy, ae, cnt = lax.fori_loop(0, _BPW, block, (zf, zi, zf))
    fs_v[...] = afy
    es_v[...] = ae
    cn_v[...] = cnt
    pltpu.sync_copy(fs_v, fsum_out.at[wid])
    pltpu.sync_copy(es_v, esum_out.at[wid])
    pltpu.sync_copy(cn_v, cnt_out.at[wid])


def kernel(score, target):
    fsum, esum, cnts = _nll_sc(score, target)
    total = jnp.sum(fsum) + jnp.float32(0.6931471805599453) * jnp.sum(
        esum).astype(jnp.float32)
    return -total / jnp.sum(cnts)


# speedup vs baseline: 3.4702x; 1.0279x over previous
import functools

import jax
import jax.numpy as jnp
from jax import lax
from jax.experimental import pallas as pl
from jax.experimental.pallas import tpu as pltpu
from jax.experimental.pallas import tpu_sc as plsc

_B, _C, _H, _W = 8, 19, 512, 512
_NW = 32
_NBLK = _B * (_H // 8) * (_W // 128)
_BPW = _NBLK // _NW
_SQRTHF_BITS = 0x3F3504F3


def _log_parts(v):
    bits = lax.bitcast_convert_type(v, jnp.int32)
    bias = bits - _SQRTHF_BITS
    e = lax.shift_right_arithmetic(bias, 23)
    m = lax.bitcast_convert_type(
        jnp.bitwise_and(bias, 0x007FFFFF) + _SQRTHF_BITS, jnp.float32)
    f = m - jnp.float32(1.0)
    z = f * f
    y = f * jnp.float32(-1.6668057665e-1) + jnp.float32(2.0000714765e-1)
    y = y * f + jnp.float32(-2.4999993993e-1)
    y = y * f + jnp.float32(3.3333331174e-1)
    y = y * f * z - jnp.float32(0.5) * z
    return f + y, e


_mesh = plsc.VectorSubcoreMesh(core_axis_name="c", subcore_axis_name="s")


@functools.partial(
    pl.kernel,
    out_type=(jax.ShapeDtypeStruct((_NW, 16), jnp.float32),
              jax.ShapeDtypeStruct((_NW, 16), jnp.int32),
              jax.ShapeDtypeStruct((_NW, 16), jnp.float32)),
    mesh=_mesh,
    scratch_types=[
        pltpu.VMEM((4 * _C * 8, 128), jnp.float32),
        pltpu.VMEM((4, 8, 128), jnp.int32),
        pltpu.VMEM((16,), jnp.float32),
        pltpu.VMEM((16,), jnp.int32),
        pltpu.VMEM((16,), jnp.float32),
        pltpu.SemaphoreType.DMA,
        pltpu.SemaphoreType.DMA,
        pltpu.SemaphoreType.DMA,
        pltpu.SemaphoreType.DMA,
    ],
    compiler_params=pltpu.CompilerParams(needs_layout_passes=False),
)
def _nll_sc(score_4d, tgt_3d, fsum_out, esum_out, cnt_out,
            cls_v, tgt_v, fs_v, es_v, cn_v, *sems):
    wid = lax.axis_index("s") * 2 + lax.axis_index("c")
    lanes = lax.broadcasted_iota(jnp.int32, (16,), 0)

    def _fire(bi, slot):
        g = wid * _BPW + bi
        b = lax.shift_right_logical(g, 8)
        h0 = lax.bitwise_and(lax.shift_right_logical(g, 2), 63) * 8
        w0 = lax.bitwise_and(g, 3) * 128
        for c in range(_C):
            pltpu.make_async_copy(
                score_4d.at[b, c, pl.ds(h0, 8), pl.ds(w0, 128)],
                cls_v.at[pl.ds((slot * _C + c) * 8, 8)], sems[slot]).start()
        pltpu.make_async_copy(
            tgt_3d.at[b, pl.ds(h0, 8), pl.ds(w0, 128)], tgt_v.at[slot],
            sems[slot]).start()

    def _drain(slot):
        pltpu.make_async_copy(
            score_4d.at[0, 0, pl.ds(0, _C * 8), pl.ds(0, 128)],
            cls_v.at[pl.ds(slot * _C * 8, _C * 8)], sems[slot]).wait()
        pltpu.make_async_copy(
            tgt_3d.at[0, pl.ds(0, 8), pl.ds(0, 128)], tgt_v.at[slot],
            sems[slot]).wait()

    def _reduce(slot, carry):
        row_base = slot * (_C * 8)

        @plsc.parallel_loop(0, 64, carry=carry, unroll=8)
        def red_body(g, c):
            afy, ae, n = c
            hl = lax.shift_right_logical(g, 3)
            wj = lax.bitwise_and(g, 7)
            t = tgt_v[slot, hl, pl.ds(wj * 16, 16)]
            v = plsc.load_gather(
                cls_v, [(row_base + hl) + t * 8, wj * 16 + lanes])
            fy, e = _log_parts(v)
            return (afy + fy, ae + e,
                    n + jnp.where(v != jnp.float32(1.0),
                                  jnp.float32(1.0), jnp.float32(0.0)))

        return red_body

    def outer(it, carry):
        for b in range(4):
            bi = it * 4 + b
            nxt = bi + 3

            @pl.when(nxt < _BPW)
            def _():
                _fire(nxt, (b + 3) % 4)

            _drain(b)
            carry = _reduce(b, carry)
        return carry

    for s in range(3):
        _fire(s, s)
    zf = jnp.zeros((16,), jnp.float32)
    zi = jnp.zeros((16,), jnp.int32)
    afy, ae, cnt = lax.fori_loop(0, _BPW // 4, outer, (zf, zi, zf))
    fs_v[...] = afy
    es_v[...] = ae
    cn_v[...] = cnt
    pltpu.sync_copy(fs_v, fsum_out.at[wid])
    pltpu.sync_copy(es_v, esum_out.at[wid])
    pltpu.sync_copy(cn_v, cnt_out.at[wid])


def kernel(score, target):
    fsum, esum, cnts = _nll_sc(score, target)
    total = jnp.sum(fsum) + jnp.float32(0.6931471805599453) * jnp.sum(
        esum).astype(jnp.float32)
    return -total / jnp.sum(cnts)
